# Initial kernel scaffold; baseline (speedup 1.0000x reference)
import jax
import jax.numpy as jnp
from jax.experimental import pallas as pl


def kernel(x, edge_index, W, b):
    raise NotImplementedError("write your pallas kernel here")


# speedup vs baseline: 18.0484x; 18.0484x over previous
import jax
import jax.numpy as jnp
from jax import lax
from jax.experimental import pallas as pl
from jax.experimental.pallas import tpu as pltpu
from jax.experimental.pallas import tpu_sc as plsc

N = 10000
E = 320000
D = 128
NC = 2
NS = 16
NW = NC * NS
EPW = E // NW
CB = 80
G = 25
NG = EPW // (G * CB)
NPAD = 10112
RPT = NPAD // NS
ZROW = N

_mesh = plsc.VectorSubcoreMesh(
    core_axis_name="c", subcore_axis_name="s", num_cores=NC, num_subcores=NS
)


def _fill_rows(ref, nrows, value):
    def row(j, _):
        def lane(k, _):
            ref[j, pl.ds(k * 16, 16)] = jnp.full((16,), value, jnp.float32)
            return 0

        return lax.fori_loop(0, D // 16, lane, 0)

    lax.fori_loop(0, nrows, row, 0)


def _zero_stripe(acc_s, zb, s):
    base = s * RPT
    for t in range(RPT // CB):
        pltpu.sync_copy(zb, acc_s.at[pl.ds(base + t * CB, CB)])
    rem = RPT % CB
    if rem:
        pltpu.sync_copy(
            zb.at[pl.ds(0, rem)], acc_s.at[pl.ds(base + (RPT // CB) * CB, rem)]
        )


def _deg_body(src_hbm, dst_hbm, srcp_hbm, degp_hbm, sv, dv, ones_v, deg_s):
    c = lax.axis_index("c")
    s = lax.axis_index("s")
    w = c * NS + s

    _fill_rows(ones_v, CB, 0.0)
    _zero_stripe(deg_s, ones_v, s)
    _fill_rows(ones_v, CB, 1.0)
    plsc.subcore_barrier()

    for g in range(NG):
        pltpu.sync_copy(src_hbm.at[w, g], sv)
        pltpu.sync_copy(dst_hbm.at[w, g], dv)

        def remap_row(j, _):
            def remap_lane(k, _):
                sl = sv[j, pl.ds(k * 16, 16)]
                dl = dv[j, pl.ds(k * 16, 16)]
                sv[j, pl.ds(k * 16, 16)] = jnp.where(
                    sl == dl, jnp.int32(ZROW), sl
                )
                return 0

            return lax.fori_loop(0, CB // 16, remap_lane, 0)

        lax.fori_loop(0, G, remap_row, 0)
        pltpu.sync_copy(sv, srcp_hbm.at[w, g])

        def scat(j, _):
            pltpu.sync_copy(ones_v, deg_s.at[sv.at[j]], add=True)
            return 0

        lax.fori_loop(0, G, scat, 0)

    plsc.subcore_barrier()
    pltpu.sync_copy(
        deg_s.at[pl.ds(s * RPT, RPT)], degp_hbm.at[c, pl.ds(s * RPT, RPT)]
    )


_deg_kernel = pl.kernel(
    _deg_body,
    out_type=(
        jax.ShapeDtypeStruct((NW, NG, G, CB), jnp.int32),
        jax.ShapeDtypeStruct((NC, NPAD, D), jnp.float32),
    ),
    mesh=_mesh,
    scratch_types=[
        pltpu.VMEM((G, CB), jnp.int32),
        pltpu.VMEM((G, CB), jnp.int32),
        pltpu.VMEM((CB, D), jnp.float32),
        pltpu.VMEM_SHARED((NPAD, D), jnp.float32),
    ],
)


def _scale_body(degp_ref, x_ref, y_ref):
    deg = degp_ref[0, :, 0:1] + degp_ref[1, :, 0:1]
    dinv = jnp.where(deg > 0.0, lax.rsqrt(deg), 0.0)
    y_ref[0:N, :] = x_ref[...] * dinv[0:N]
    y_ref[N:NPAD, :] = jnp.zeros((NPAD - N, D), jnp.float32)


def _agg_body(y_hbm, srcp_hbm, dst_hbm, sp_hbm, sv, dv, buf, acc_s, sem):
    c = lax.axis_index("c")
    s = lax.axis_index("s")
    w = c * NS + s

    _fill_rows(buf, CB, 0.0)
    _zero_stripe(acc_s, buf, s)
    plsc.subcore_barrier()

    for g in range(NG):
        pltpu.sync_copy(srcp_hbm.at[w, g], sv)
        pltpu.sync_copy(dst_hbm.at[w, g], dv)

        def blk(j, _):
            pltpu.async_copy(y_hbm.at[sv.at[j]], buf, sem).wait()
            pltpu.sync_copy(buf, acc_s.at[dv.at[j]], add=True)
            return 0

        lax.fori_loop(0, G, blk, 0)

    plsc.subcore_barrier()
    pltpu.sync_copy(
        acc_s.at[pl.ds(s * RPT, RPT)], sp_hbm.at[c, pl.ds(s * RPT, RPT)]
    )


_agg_kernel = pl.kernel(
    _agg_body,
    out_type=jax.ShapeDtypeStruct((NC, NPAD, D), jnp.float32),
    mesh=_mesh,
    scratch_types=[
        pltpu.VMEM((G, CB), jnp.int32),
        pltpu.VMEM((G, CB), jnp.int32),
        pltpu.VMEM((CB, D), jnp.float32),
        pltpu.VMEM_SHARED((NPAD, D), jnp.float32),
        pltpu.SemaphoreType.DMA,
    ],
)


def _out_body(x_ref, degp_ref, sp_ref, w_ref, b_ref, o_ref):
    deg = degp_ref[0, 0:N, 0:1] + degp_ref[1, 0:N, 0:1]
    dinv = jnp.where(deg > 0.0, lax.rsqrt(deg), 0.0)
    z = dinv * (sp_ref[0, 0:N, :] + sp_ref[1, 0:N, :])
    out = jnp.dot(x_ref[...], w_ref[0], preferred_element_type=jnp.float32)
    out = out - jnp.dot(z, w_ref[1], preferred_element_type=jnp.float32)
    o_ref[...] = out + b_ref[0]


def kernel(x, edge_index, W, b):
    src4 = edge_index[0].reshape(NW, NG, G, CB)
    dst4 = edge_index[1].reshape(NW, NG, G, CB)

    srcp, degp = _deg_kernel(src4, dst4)

    y = pl.pallas_call(
        _scale_body,
        out_shape=jax.ShapeDtypeStruct((NPAD, D), jnp.float32),
    )(degp, x)

    sp = _agg_kernel(y, srcp, dst4)

    out = pl.pallas_call(
        _out_body,
        out_shape=jax.ShapeDtypeStruct((N, D), jnp.float32),
    )(x, degp, sp, W, b.reshape(1, D))
    return out


# speedup vs baseline: 21.5415x; 1.1935x over previous
import jax
import jax.numpy as jnp
from jax import lax
from jax.experimental import pallas as pl
from jax.experimental.pallas import tpu as pltpu
from jax.experimental.pallas import tpu_sc as plsc

N = 10000
E = 320000
D = 128
NC = 2
NS = 16
NW = NC * NS
EPW = E // NW
CB = 80
G = 25
NG = EPW // (G * CB)
NPAD = 10112
RPT = NPAD // NS
ZROW = N

_mesh = plsc.VectorSubcoreMesh(
    core_axis_name="c", subcore_axis_name="s", num_cores=NC, num_subcores=NS
)


def _fill_rows(ref, nrows, value):
    def row(j, _):
        def lane(k, _):
            ref[j, pl.ds(k * 16, 16)] = jnp.full((16,), value, jnp.float32)
            return 0

        return lax.fori_loop(0, D // 16, lane, 0)

    lax.fori_loop(0, nrows, row, 0)


def _zero_stripe(acc_s, zb, s):
    base = s * RPT
    for t in range(RPT // CB):
        pltpu.sync_copy(zb, acc_s.at[pl.ds(base + t * CB, CB)])
    rem = RPT % CB
    if rem:
        pltpu.sync_copy(
            zb.at[pl.ds(0, rem)], acc_s.at[pl.ds(base + (RPT // CB) * CB, rem)]
        )


def _deg_body(src_hbm, dst_hbm, srcp_hbm, degp_hbm, sv0, sv1, dv, ones_v,
              deg_s, sem_s):
    c = lax.axis_index("c")
    s = lax.axis_index("s")
    w = c * NS + s
    svs = [sv0, sv1]

    _fill_rows(ones_v, CB, 0.0)
    _zero_stripe(deg_s, ones_v, s)
    _fill_rows(ones_v, CB, 1.0)
    plsc.subcore_barrier()

    def stage_remap(g, sv):
        pltpu.sync_copy(src_hbm.at[w, g], sv)
        pltpu.sync_copy(dst_hbm.at[w, g], dv)

        def remap_row(j, _):
            def remap_lane(k, _):
                sl = sv[j, pl.ds(k * 16, 16)]
                dl = dv[j, pl.ds(k * 16, 16)]
                sv[j, pl.ds(k * 16, 16)] = jnp.where(
                    sl == dl, jnp.int32(ZROW), sl
                )
                return 0

            return lax.fori_loop(0, CB // 16, remap_lane, 0)

        lax.fori_loop(0, G, remap_row, 0)
        pltpu.sync_copy(sv, srcp_hbm.at[w, g])

    stage_remap(0, svs[0])
    for g in range(NG):
        sv = svs[g % 2]

        def fire(j, _):
            pltpu.async_copy(ones_v, deg_s.at[sv.at[j]], sem_s, add=True)
            return 0

        lax.fori_loop(0, G, fire, 0)
        if g + 1 < NG:
            stage_remap(g + 1, svs[(g + 1) % 2])

        def drain(j, _):
            pltpu.make_async_copy(ones_v, deg_s.at[sv.at[j]], sem_s).wait()
            return 0

        lax.fori_loop(0, G, drain, 0)

    plsc.subcore_barrier()
    pltpu.sync_copy(
        deg_s.at[pl.ds(s * RPT, RPT)], degp_hbm.at[c, pl.ds(s * RPT, RPT)]
    )


_deg_kernel = pl.kernel(
    _deg_body,
    out_type=(
        jax.ShapeDtypeStruct((NW, NG, G, CB), jnp.int32),
        jax.ShapeDtypeStruct((NC, NPAD, D), jnp.float32),
    ),
    mesh=_mesh,
    scratch_types=[
        pltpu.VMEM((G, CB), jnp.int32),
        pltpu.VMEM((G, CB), jnp.int32),
        pltpu.VMEM((G, CB), jnp.int32),
        pltpu.VMEM((CB, D), jnp.float32),
        pltpu.VMEM_SHARED((NPAD, D), jnp.float32),
        pltpu.SemaphoreType.DMA,
    ],
)


def _scale_body(degp_ref, x_ref, y_ref):
    deg = degp_ref[0, :, 0:1] + degp_ref[1, :, 0:1]
    dinv = jnp.where(deg > 0.0, lax.rsqrt(deg), 0.0)
    y_ref[0:N, :] = x_ref[...] * dinv[0:N]
    y_ref[N:NPAD, :] = jnp.zeros((NPAD - N, D), jnp.float32)


def _agg_body(y_hbm, srcp_hbm, dst_hbm, sp_hbm, sv, dv, buf_a, buf_b, acc_s,
              sem_ga, sem_gb, sem_sa, sem_sb):
    c = lax.axis_index("c")
    s = lax.axis_index("s")
    w = c * NS + s

    _fill_rows(buf_a, CB, 0.0)
    _zero_stripe(acc_s, buf_a, s)
    plsc.subcore_barrier()

    def gstart(buf, sem, j):
        pltpu.async_copy(y_hbm.at[sv.at[j]], buf, sem)

    def gwait(buf, sem, j):
        pltpu.make_async_copy(y_hbm.at[sv.at[j]], buf, sem).wait()

    def sstart(buf, sem, j):
        pltpu.async_copy(buf, acc_s.at[dv.at[j]], sem, add=True)

    def swait(buf, sem, j):
        pltpu.make_async_copy(buf, acc_s.at[dv.at[j]], sem).wait()

    for g in range(NG):
        pltpu.sync_copy(srcp_hbm.at[w, g], sv)
        pltpu.sync_copy(dst_hbm.at[w, g], dv)

        gstart(buf_a, sem_ga, 0)
        gwait(buf_a, sem_ga, 0)
        gstart(buf_b, sem_gb, 1)
        sstart(buf_a, sem_sa, 0)

        def pair(p, _):
            j1 = 2 * p + 1
            j2 = 2 * p + 2
            gwait(buf_b, sem_gb, j1)
            swait(buf_a, sem_sa, j1 - 1)
            gstart(buf_a, sem_ga, j1 + 1)
            sstart(buf_b, sem_sb, j1)
            gwait(buf_a, sem_ga, j2)
            swait(buf_b, sem_sb, j2 - 1)
            gstart(buf_b, sem_gb, j2 + 1)
            sstart(buf_a, sem_sa, j2)
            return 0

        lax.fori_loop(0, (G - 3) // 2, pair, 0)

        gwait(buf_b, sem_gb, G - 2)
        swait(buf_a, sem_sa, G - 3)
        gstart(buf_a, sem_ga, G - 1)
        sstart(buf_b, sem_sb, G - 2)
        gwait(buf_a, sem_ga, G - 1)
        swait(buf_b, sem_sb, G - 2)
        sstart(buf_a, sem_sa, G - 1)
        swait(buf_a, sem_sa, G - 1)

    plsc.subcore_barrier()
    pltpu.sync_copy(
        acc_s.at[pl.ds(s * RPT, RPT)], sp_hbm.at[c, pl.ds(s * RPT, RPT)]
    )


_agg_kernel = pl.kernel(
    _agg_body,
    out_type=jax.ShapeDtypeStruct((NC, NPAD, D), jnp.float32),
    mesh=_mesh,
    scratch_types=[
        pltpu.VMEM((G, CB), jnp.int32),
        pltpu.VMEM((G, CB), jnp.int32),
        pltpu.VMEM((CB, D), jnp.float32),
        pltpu.VMEM((CB, D), jnp.float32),
        pltpu.VMEM_SHARED((NPAD, D), jnp.float32),
        pltpu.SemaphoreType.DMA,
        pltpu.SemaphoreType.DMA,
        pltpu.SemaphoreType.DMA,
        pltpu.SemaphoreType.DMA,
    ],
)


def _mm0_body(x_ref, w_ref, b_ref, o_ref):
    out = jnp.dot(x_ref[...], w_ref[0], preferred_element_type=jnp.float32)
    o_ref[...] = out + b_ref[0]


def _out_body(out0_ref, degp_ref, sp_ref, w_ref, o_ref):
    deg = degp_ref[0, 0:N, 0:1] + degp_ref[1, 0:N, 0:1]
    dinv = jnp.where(deg > 0.0, lax.rsqrt(deg), 0.0)
    z = dinv * (sp_ref[0, 0:N, :] + sp_ref[1, 0:N, :])
    o_ref[...] = out0_ref[...] - jnp.dot(
        z, w_ref[1], preferred_element_type=jnp.float32
    )


def kernel(x, edge_index, W, b):
    src4 = edge_index[0].reshape(NW, NG, G, CB)
    dst4 = edge_index[1].reshape(NW, NG, G, CB)

    out0 = pl.pallas_call(
        _mm0_body,
        out_shape=jax.ShapeDtypeStruct((N, D), jnp.float32),
    )(x, W, b.reshape(1, D))

    srcp, degp = _deg_kernel(src4, dst4)

    y = pl.pallas_call(
        _scale_body,
        out_shape=jax.ShapeDtypeStruct((NPAD, D), jnp.float32),
    )(degp, x)

    sp = _agg_kernel(y, srcp, dst4)

    out = pl.pallas_call(
        _out_body,
        out_shape=jax.ShapeDtypeStruct((N, D), jnp.float32),
    )(out0, degp, sp, W)
    return out


# speedup vs baseline: 25.0476x; 1.1628x over previous
import jax
import jax.numpy as jnp
from jax import lax
from jax.experimental import pallas as pl
from jax.experimental.pallas import tpu as pltpu
from jax.experimental.pallas import tpu_sc as plsc

N = 10000
E = 320000
D = 128
NC = 2
NS = 16
NW = NC * NS
EPW = E // NW
CB = 80
G = 25
NG = EPW // (G * CB)
NPAD = 10112
RPT = NPAD // NS
ZROW = N

_mesh = plsc.VectorSubcoreMesh(
    core_axis_name="c", subcore_axis_name="s", num_cores=NC, num_subcores=NS
)


def _fill_rows(ref, nrows, value):
    def row(j, _):
        def lane(k, _):
            ref[j, pl.ds(k * 16, 16)] = jnp.full((16,), value, jnp.float32)
            return 0

        return lax.fori_loop(0, D // 16, lane, 0)

    lax.fori_loop(0, nrows, row, 0)


def _zero_stripe(acc_s, zb, s):
    base = s * RPT
    for t in range(RPT // CB):
        pltpu.sync_copy(zb, acc_s.at[pl.ds(base + t * CB, CB)])
    rem = RPT % CB
    if rem:
        pltpu.sync_copy(
            zb.at[pl.ds(0, rem)], acc_s.at[pl.ds(base + (RPT // CB) * CB, rem)]
        )


def _deg_body(src_hbm, dst_hbm, srcp_hbm, degp_hbm, sv0, sv1, dv, ones_v,
              deg_s, sem_s):
    c = lax.axis_index("c")
    s = lax.axis_index("s")
    w = c * NS + s
    svs = [sv0, sv1]

    _fill_rows(ones_v, CB, 0.0)
    _zero_stripe(deg_s, ones_v, s)
    _fill_rows(ones_v, CB, 1.0)
    plsc.subcore_barrier()

    def stage_remap(g, sv):
        pltpu.sync_copy(src_hbm.at[w, g], sv)
        pltpu.sync_copy(dst_hbm.at[w, g], dv)

        def remap_row(j, _):
            def remap_lane(k, _):
                sl = sv[j, pl.ds(k * 16, 16)]
                dl = dv[j, pl.ds(k * 16, 16)]
                sv[j, pl.ds(k * 16, 16)] = jnp.where(
                    sl == dl, jnp.int32(ZROW), sl
                )
                return 0

            return lax.fori_loop(0, CB // 16, remap_lane, 0)

        lax.fori_loop(0, G, remap_row, 0)
        pltpu.sync_copy(sv, srcp_hbm.at[w, g])

    stage_remap(0, svs[0])
    for g in range(NG):
        sv = svs[g % 2]

        def fire(j, _):
            pltpu.async_copy(ones_v, deg_s.at[sv.at[j]], sem_s, add=True)
            return 0

        lax.fori_loop(0, G, fire, 0)
        if g + 1 < NG:
            stage_remap(g + 1, svs[(g + 1) % 2])

        def drain(j, _):
            pltpu.make_async_copy(ones_v, deg_s.at[sv.at[j]], sem_s).wait()
            return 0

        lax.fori_loop(0, G, drain, 0)

    plsc.subcore_barrier()
    pltpu.sync_copy(
        deg_s.at[pl.ds(s * RPT, RPT)], degp_hbm.at[c, pl.ds(s * RPT, RPT)]
    )


_deg_kernel = pl.kernel(
    _deg_body,
    out_type=(
        jax.ShapeDtypeStruct((NW, NG, G, CB), jnp.int32),
        jax.ShapeDtypeStruct((NC, NPAD, D), jnp.float32),
    ),
    mesh=_mesh,
    scratch_types=[
        pltpu.VMEM((G, CB), jnp.int32),
        pltpu.VMEM((G, CB), jnp.int32),
        pltpu.VMEM((G, CB), jnp.int32),
        pltpu.VMEM((CB, D), jnp.float32),
        pltpu.VMEM_SHARED((NPAD, D), jnp.float32),
        pltpu.SemaphoreType.DMA,
    ],
)


def _scale_body(degp_ref, x_ref, y_ref):
    deg = degp_ref[0, :, 0:1] + degp_ref[1, :, 0:1]
    dinv = jnp.where(deg > 0.0, lax.rsqrt(deg), 0.0)
    y_ref[0:N, :] = x_ref[...] * dinv[0:N]
    y_ref[N:NPAD, :] = jnp.zeros((NPAD - N, D), jnp.float32)


def _agg_body(y_hbm, srcp_hbm, dst_hbm, sp_hbm, sv, dv, b0, b1, b2, b3,
              acc_s, sg0, sg1, sg2, sg3, ss0, ss1, ss2, ss3):
    c = lax.axis_index("c")
    s = lax.axis_index("s")
    w = c * NS + s
    bufs = [b0, b1, b2, b3]
    sgs = [sg0, sg1, sg2, sg3]
    sss = [ss0, ss1, ss2, ss3]

    _fill_rows(b0, CB, 0.0)
    _zero_stripe(acc_s, b0, s)
    plsc.subcore_barrier()

    def gstart(j, b):
        pltpu.async_copy(y_hbm.at[sv.at[j]], bufs[b], sgs[b])

    def gwait(j, b):
        pltpu.make_async_copy(y_hbm.at[sv.at[j]], bufs[b], sgs[b]).wait()

    def sstart(j, b):
        pltpu.async_copy(bufs[b], acc_s.at[dv.at[j]], sss[b], add=True)

    def swait(j, b):
        pltpu.make_async_copy(bufs[b], acc_s.at[dv.at[j]], sss[b]).wait()

    for g in range(NG):
        pltpu.sync_copy(srcp_hbm.at[w, g], sv)
        pltpu.sync_copy(dst_hbm.at[w, g], dv)

        gstart(0, 0)
        gstart(1, 1)
        gwait(0, 0); sstart(0, 0); gstart(2, 2)
        gwait(1, 1); sstart(1, 1); gstart(3, 3)
        gwait(2, 2); sstart(2, 2); swait(0, 0); gstart(4, 0)

        def quad(p, _):
            j = 4 * p + 3
            gwait(j, 3); sstart(j, 3); swait(j - 2, 1); gstart(j + 2, 1)
            gwait(j + 1, 0); sstart(j + 1, 0); swait(j - 1, 2); gstart(j + 3, 2)
            gwait(j + 2, 1); sstart(j + 2, 1); swait(j, 3); gstart(j + 4, 3)
            gwait(j + 3, 2); sstart(j + 3, 2); swait(j + 1, 0); gstart(j + 5, 0)
            return 0

        lax.fori_loop(0, (G - 5) // 4, quad, 0)

        gwait(G - 2, 3); sstart(G - 2, 3); swait(G - 4, 1)
        gwait(G - 1, 0); sstart(G - 1, 0); swait(G - 3, 2)
        swait(G - 2, 3)
        swait(G - 1, 0)

    plsc.subcore_barrier()
    pltpu.sync_copy(
        acc_s.at[pl.ds(s * RPT, RPT)], sp_hbm.at[c, pl.ds(s * RPT, RPT)]
    )


_agg_kernel = pl.kernel(
    _agg_body,
    out_type=jax.ShapeDtypeStruct((NC, NPAD, D), jnp.float32),
    mesh=_mesh,
    scratch_types=[
        pltpu.VMEM((G, CB), jnp.int32),
        pltpu.VMEM((G, CB), jnp.int32),
        pltpu.VMEM((CB, D), jnp.float32),
        pltpu.VMEM((CB, D), jnp.float32),
        pltpu.VMEM((CB, D), jnp.float32),
        pltpu.VMEM((CB, D), jnp.float32),
        pltpu.VMEM_SHARED((NPAD, D), jnp.float32),
        pltpu.SemaphoreType.DMA,
        pltpu.SemaphoreType.DMA,
        pltpu.SemaphoreType.DMA,
        pltpu.SemaphoreType.DMA,
        pltpu.SemaphoreType.DMA,
        pltpu.SemaphoreType.DMA,
        pltpu.SemaphoreType.DMA,
        pltpu.SemaphoreType.DMA,
    ],
)


def _mm0_body(x_ref, w_ref, b_ref, o_ref):
    out = jnp.dot(x_ref[...], w_ref[0], preferred_element_type=jnp.float32)
    o_ref[...] = out + b_ref[0]


def _out_body(out0_ref, degp_ref, sp_ref, w_ref, o_ref):
    deg = degp_ref[0, 0:N, 0:1] + degp_ref[1, 0:N, 0:1]
    dinv = jnp.where(deg > 0.0, lax.rsqrt(deg), 0.0)
    z = dinv * (sp_ref[0, 0:N, :] + sp_ref[1, 0:N, :])
    o_ref[...] = out0_ref[...] - jnp.dot(
        z, w_ref[1], preferred_element_type=jnp.float32
    )


def kernel(x, edge_index, W, b):
    src4 = edge_index[0].reshape(NW, NG, G, CB)
    dst4 = edge_index[1].reshape(NW, NG, G, CB)

    out0 = pl.pallas_call(
        _mm0_body,
        out_shape=jax.ShapeDtypeStruct((N, D), jnp.float32),
    )(x, W, b.reshape(1, D))

    srcp, degp = _deg_kernel(src4, dst4)

    y = pl.pallas_call(
        _scale_body,
        out_shape=jax.ShapeDtypeStruct((NPAD, D), jnp.float32),
    )(degp, x)

    sp = _agg_kernel(y, srcp, dst4)

    out = pl.pallas_call(
        _out_body,
        out_shape=jax.ShapeDtypeStruct((N, D), jnp.float32),
    )(out0, degp, sp, W)
    return out
